# baseline (speedup 1.0000x reference)
import functools

import jax
import jax.numpy as jnp
from jax import lax
from jax.experimental import pallas as pl
from jax.experimental.pallas import tpu as pltpu
from jax.experimental.pallas import tpu_sc as plsc

_HIDDEN = 1024
_NC = 2
_NS = 16
_NW = _NC * _NS
_CHUNK = 32
_NBUF = 3
_L = 16


@functools.cache
def _sc_lookup(total: int, n_rows: int):
    per_w = total // _NW
    nch = per_w // _CHUNK
    nhalf = _CHUNK // _L
    nblk = _HIDDEN // _L
    mesh = plsc.VectorSubcoreMesh(core_axis_name="c", subcore_axis_name="s")

    @functools.partial(
        pl.kernel,
        out_type=jax.ShapeDtypeStruct((total, _HIDDEN), jnp.float32),
        mesh=mesh,
        compiler_params=pltpu.CompilerParams(use_tc_tiling_on_sc=False, needs_layout_passes=False),
        scratch_types=[
            pltpu.VMEM((per_w,), jnp.int32),
            pltpu.VMEM((n_rows, _HIDDEN), jnp.float32),
            *[pltpu.VMEM((_CHUNK, _HIDDEN), jnp.float32) for _ in range(_NBUF)],
            pltpu.SemaphoreType.DMA,
            *[pltpu.SemaphoreType.DMA for _ in range(_NBUF)],
        ],
    )
    def k(ids_hbm, table_hbm, out_hbm, idx_v, table_v, *rest):
        bufs = rest[:_NBUF]
        gsem = rest[_NBUF]
        ssems = rest[_NBUF + 1:]
        wid = lax.axis_index("s") * _NC + lax.axis_index("c")
        base = wid * per_w
        cp_t = pltpu.async_copy(table_hbm, table_v, gsem)
        pltpu.sync_copy(ids_hbm.at[wid], idx_v)
        cp_t.wait()
        iota = lax.iota(jnp.int32, _L)
        scp = [None] * nch
        for c in range(nch):
            slot = c % _NBUF
            if c >= _NBUF:
                scp[c - _NBUF].wait()
            buf = bufs[slot]
            for h in range(nhalf):
                ids_h = idx_v[pl.ds((c * nhalf + h) * _L, _L)]
                dnums = lax.GatherDimensionNumbers(
                    offset_dims=(), collapsed_slice_dims=(0,),
                    start_index_map=(0,))
                basevecs = [
                    lax.gather(
                        ids_h,
                        jnp.full((_L, 1), j, jnp.int32),
                        dnums, (1,),
                        mode=lax.GatherScatterMode.PROMISE_IN_BOUNDS,
                    )
                    for j in range(_L)
                ]

                def blk_body(t, _, basevecs=basevecs, buf=buf, h=h):
                    off = t * _L
                    colv = iota + off
                    for j in range(_L):
                        val = plsc.load_gather(table_v, [basevecs[j], colv])
                        buf[h * _L + j, pl.ds(off, _L)] = val
                    return ()

                lax.fori_loop(0, nblk, blk_body, (), unroll=False)
            scp[c] = pltpu.async_copy(
                buf,
                out_hbm.at[pl.ds(base + c * _CHUNK, _CHUNK)],
                ssems[slot],
            )
        for c in range(nch - _NBUF, nch):
            scp[c].wait()

    return k


def kernel(ids, indicator_embs):
    b, s = ids.shape
    total = b * s
    ids_w = ids.astype(jnp.int32).reshape(_NW, total // _NW)
    out = _sc_lookup(total, indicator_embs.shape[0])(
        ids_w, indicator_embs
    )
    return out.reshape(b, s, _HIDDEN)


# speedup vs baseline: 1.8719x; 1.8719x over previous
import functools

import jax
import jax.numpy as jnp
from jax import lax
from jax.experimental import pallas as pl
from jax.experimental.pallas import tpu as pltpu
from jax.experimental.pallas import tpu_sc as plsc

_HIDDEN = 1024
_NC = 2
_NS = 16
_NW = _NC * _NS
_CHUNK = 32
_NBUF = 3
_L = 16


@functools.cache
def _sc_lookup(total: int, n_rows: int):
    per_w = total // _NW
    nch = per_w // _CHUNK
    nhalf = _CHUNK // _L
    nblk = _HIDDEN // _L
    mesh = plsc.VectorSubcoreMesh(core_axis_name="c", subcore_axis_name="s")

    @functools.partial(
        pl.kernel,
        out_type=jax.ShapeDtypeStruct((total, _HIDDEN), jnp.float32),
        mesh=mesh,
        compiler_params=pltpu.CompilerParams(use_tc_tiling_on_sc=False, needs_layout_passes=False),
        scratch_types=[
            pltpu.VMEM((per_w,), jnp.int32),
            pltpu.VMEM((n_rows, _HIDDEN), jnp.float32),
            *[pltpu.VMEM((_CHUNK, _HIDDEN), jnp.float32) for _ in range(_NBUF)],
            pltpu.SemaphoreType.DMA,
            *[pltpu.SemaphoreType.DMA for _ in range(_NBUF)],
        ],
    )
    def k(ids_hbm, table_hbm, out_hbm, idx_v, table_v, *rest):
        bufs = rest[:_NBUF]
        gsem = rest[_NBUF]
        ssems = rest[_NBUF + 1:]
        wid = lax.axis_index("s") * _NC + lax.axis_index("c")
        base = wid * per_w
        cp_t = pltpu.async_copy(table_hbm, table_v, gsem)
        pltpu.sync_copy(ids_hbm.at[wid], idx_v)
        cp_t.wait()
        iota = lax.iota(jnp.int32, _L)
        scp = [None] * nch
        for c in range(nch):
            slot = c % _NBUF
            if c >= _NBUF:
                scp[c - _NBUF].wait()
            buf = bufs[slot]
            for h in range(nhalf):
                ids_h = idx_v[pl.ds((c * nhalf + h) * _L, _L)]
                dnums = lax.GatherDimensionNumbers(
                    offset_dims=(), collapsed_slice_dims=(0,),
                    start_index_map=(0,))
                basevecs = [
                    lax.gather(
                        ids_h,
                        jnp.full((_L, 1), j, jnp.int32),
                        dnums, (1,),
                        mode=lax.GatherScatterMode.PROMISE_IN_BOUNDS,
                    )
                    for j in range(_L)
                ]

                @plsc.parallel_loop(0, nblk, 1, unroll=2)
                def blk_body(t, buf=buf, h=h, basevecs=basevecs):
                    off = t * _L
                    colv = iota + off
                    for j in range(_L):
                        val = plsc.load_gather(table_v, [basevecs[j], colv])
                        buf[h * _L + j, pl.ds(off, _L)] = val
            scp[c] = pltpu.async_copy(
                buf,
                out_hbm.at[pl.ds(base + c * _CHUNK, _CHUNK)],
                ssems[slot],
            )
        for c in range(nch - _NBUF, nch):
            scp[c].wait()

    return k


def kernel(ids, indicator_embs):
    b, s = ids.shape
    total = b * s
    ids_w = ids.astype(jnp.int32).reshape(_NW, total // _NW)
    out = _sc_lookup(total, indicator_embs.shape[0])(
        ids_w, indicator_embs
    )
    return out.reshape(b, s, _HIDDEN)


# speedup vs baseline: 1.9269x; 1.0294x over previous
import functools

import jax
import jax.numpy as jnp
from jax import lax
from jax.experimental import pallas as pl
from jax.experimental.pallas import tpu as pltpu
from jax.experimental.pallas import tpu_sc as plsc

_HIDDEN = 1024
_NC = 2
_NS = 16
_NW = _NC * _NS
_CHUNK = 32
_NBUF = 3
_L = 16


@functools.cache
def _sc_lookup(total: int, n_rows: int):
    per_w = total // _NW
    nch = per_w // _CHUNK
    nhalf = _CHUNK // _L
    nblk = _HIDDEN // _L
    mesh = plsc.VectorSubcoreMesh(core_axis_name="c", subcore_axis_name="s")

    @functools.partial(
        pl.kernel,
        out_type=jax.ShapeDtypeStruct((total, _HIDDEN), jnp.float32),
        mesh=mesh,
        compiler_params=pltpu.CompilerParams(use_tc_tiling_on_sc=False, needs_layout_passes=False),
        scratch_types=[
            pltpu.VMEM((per_w,), jnp.int32),
            pltpu.VMEM((n_rows, _HIDDEN), jnp.float32),
            *[pltpu.VMEM((_CHUNK, _HIDDEN), jnp.float32) for _ in range(_NBUF)],
            pltpu.SemaphoreType.DMA,
            *[pltpu.SemaphoreType.DMA for _ in range(_NBUF)],
        ],
    )
    def k(ids_hbm, table_hbm, out_hbm, idx_v, table_v, *rest):
        bufs = rest[:_NBUF]
        gsem = rest[_NBUF]
        ssems = rest[_NBUF + 1:]
        wid = lax.axis_index("s") * _NC + lax.axis_index("c")
        base = wid * per_w
        cp_t = pltpu.async_copy(table_hbm, table_v, gsem)
        pltpu.sync_copy(ids_hbm.at[wid], idx_v)
        cp_t.wait()
        iota = lax.iota(jnp.int32, _L)
        scp = [None] * nch
        for c in range(nch):
            slot = c % _NBUF
            if c >= _NBUF:
                scp[c - _NBUF].wait()
            buf = bufs[slot]

            def grp_body(g, _, buf=buf, c=c):
                ids_h = idx_v[pl.ds(c * _CHUNK + g * _L, _L)]
                dnums = lax.GatherDimensionNumbers(
                    offset_dims=(), collapsed_slice_dims=(0,),
                    start_index_map=(0,))
                basevecs = [
                    lax.gather(
                        ids_h,
                        jnp.full((_L, 1), j, jnp.int32),
                        dnums, (1,),
                        mode=lax.GatherScatterMode.PROMISE_IN_BOUNDS,
                    )
                    for j in range(_L)
                ]

                @plsc.parallel_loop(0, nblk, 1, unroll=4)
                def blk_body(t, buf=buf, g=g, basevecs=basevecs):
                    off = t * _L
                    colv = iota + off
                    for j in range(_L):
                        val = plsc.load_gather(table_v, [basevecs[j], colv])
                        buf[g * _L + j, pl.ds(off, _L)] = val
                return ()

            lax.fori_loop(0, nhalf, grp_body, (), unroll=False)
            scp[c] = pltpu.async_copy(
                buf,
                out_hbm.at[pl.ds(base + c * _CHUNK, _CHUNK)],
                ssems[slot],
            )
        for c in range(nch - _NBUF, nch):
            scp[c].wait()

    return k


def kernel(ids, indicator_embs):
    b, s = ids.shape
    total = b * s
    ids_w = ids.astype(jnp.int32).reshape(_NW, total // _NW)
    out = _sc_lookup(total, indicator_embs.shape[0])(
        ids_w, indicator_embs
    )
    return out.reshape(b, s, _HIDDEN)
